# Initial kernel scaffold; baseline (speedup 1.0000x reference)
import jax
import jax.numpy as jnp
from jax.experimental import pallas as pl


def kernel(features, edge_index, W_neigh, b):
    raise NotImplementedError("write your pallas kernel here")


# speedup vs baseline: 9.0698x; 9.0698x over previous
import functools

import jax
import jax.numpy as jnp
from jax import lax
from jax.experimental import pallas as pl
from jax.experimental.pallas import tpu as pltpu
from jax.experimental.pallas import tpu_sc as plsc

NW = 32
CHUNK = 128


def _fuse_tc(a0_ref, a1_ref, x_ref, d0_ref, d1_ref, w_ref, b_ref, o_ref):
    h = a0_ref[...] + a1_ref[...] + x_ref[...]
    deg = d0_ref[...] + d1_ref[...] + 1.0
    p = jnp.dot(h, w_ref[...], preferred_element_type=jnp.float32)
    o_ref[...] = p / deg + b_ref[...]


def _make_sc_scatter(n_pad, chunks_per_w, f):
    rows_per_tile = n_pad // 16
    mesh = plsc.VectorSubcoreMesh(core_axis_name="c", subcore_axis_name="s")

    @functools.partial(
        pl.kernel,
        mesh=mesh,
        out_type=[
            jax.ShapeDtypeStruct((2, n_pad, f), jnp.float32),
            jax.ShapeDtypeStruct((2 * n_pad,), jnp.float32),
        ],
        scratch_types=[
            pltpu.VMEM((chunks_per_w, CHUNK), jnp.int32),
            pltpu.VMEM((chunks_per_w, CHUNK), jnp.int32),
            pltpu.VMEM((CHUNK, f), jnp.float32),
            pltpu.VMEM((CHUNK,), jnp.float32),
            pltpu.VMEM_SHARED((n_pad, f), jnp.float32),
            pltpu.VMEM_SHARED((n_pad,), jnp.float32),
            pltpu.SemaphoreType.DMA,
        ],
    )
    def sc_scatter(src_hbm, dst_hbm, x_hbm, z2_hbm, z1_hbm,
                   acc_out, deg_out,
                   src_v, dst_v, buf_v, ones_v, acc_sh, deg_sh, sem):
        c = lax.axis_index("c")
        s = lax.axis_index("s")
        wid = c * 16 + s
        row0 = s * rows_per_tile
        sl = pl.ds(row0, rows_per_tile)

        pltpu.sync_copy(z2_hbm.at[sl], acc_sh.at[sl])
        pltpu.sync_copy(z1_hbm.at[sl], deg_sh.at[sl])

        pltpu.sync_copy(src_hbm.at[wid], src_v)
        pltpu.sync_copy(dst_hbm.at[wid], dst_v)
        for i in range(CHUNK // 16):
            ones_v[pl.ds(i * 16, 16)] = jnp.ones((16,), jnp.float32)

        plsc.subcore_barrier()

        def body(j, carry):
            pltpu.async_copy(x_hbm.at[src_v.at[j]], buf_v, sem).wait()
            pltpu.sync_copy(buf_v, acc_sh.at[dst_v.at[j]], add=True)
            pltpu.sync_copy(ones_v, deg_sh.at[dst_v.at[j]], add=True)
            return carry

        lax.fori_loop(0, chunks_per_w, body, 0)

        plsc.subcore_barrier()

        pltpu.sync_copy(acc_sh.at[sl], acc_out.at[c, sl])
        pltpu.sync_copy(deg_sh.at[sl],
                        deg_out.at[pl.ds(c * n_pad + row0, rows_per_tile)])

    return sc_scatter


def kernel(features, edge_index, W_neigh, b):
    n, f = features.shape
    d = W_neigh.shape[1]
    e = edge_index.shape[1]

    n_pad = (n // 2048 + 1) * 2048
    chunks_per_w = -(-e // (NW * CHUNK))
    e_pad = NW * chunks_per_w * CHUNK

    src = edge_index[0].astype(jnp.int32)
    dst = edge_index[1].astype(jnp.int32)
    pad = e_pad - e
    spread = jnp.arange(pad, dtype=jnp.int32)
    src_p = jnp.concatenate([src, spread % n])
    dst_p = jnp.concatenate([dst, n + spread % (n_pad - n)])
    src_r = src_p.reshape(NW, chunks_per_w, CHUNK)
    dst_r = dst_p.reshape(NW, chunks_per_w, CHUNK)
    z2 = jnp.zeros((n_pad, f), jnp.float32)
    z1 = jnp.zeros((n_pad,), jnp.float32)

    acc_part, deg_part = _make_sc_scatter(n_pad, chunks_per_w, f)(
        src_r, dst_r, features, z2, z1)

    blk = 1000
    grid = n // blk
    a0 = acc_part[0, :n]
    a1 = acc_part[1, :n]
    d0 = deg_part[:n].reshape(n, 1)
    d1 = deg_part[n_pad:n_pad + n].reshape(n, 1)
    b2 = b.reshape(1, d)
    out = pl.pallas_call(
        _fuse_tc,
        grid=(grid,),
        in_specs=[
            pl.BlockSpec((blk, f), lambda i: (i, 0)),
            pl.BlockSpec((blk, f), lambda i: (i, 0)),
            pl.BlockSpec((blk, f), lambda i: (i, 0)),
            pl.BlockSpec((blk, 1), lambda i: (i, 0)),
            pl.BlockSpec((blk, 1), lambda i: (i, 0)),
            pl.BlockSpec((f, d), lambda i: (0, 0)),
            pl.BlockSpec((1, d), lambda i: (0, 0)),
        ],
        out_specs=pl.BlockSpec((blk, d), lambda i: (i, 0)),
        out_shape=jax.ShapeDtypeStruct((n, d), jnp.float32),
    )(a0, a1, features, d0, d1, W_neigh, b2)
    return out


# speedup vs baseline: 13.9494x; 1.5380x over previous
import functools

import jax
import jax.numpy as jnp
from jax import lax
from jax.experimental import pallas as pl
from jax.experimental.pallas import tpu as pltpu
from jax.experimental.pallas import tpu_sc as plsc

NW = 32
CHUNK = 128


def _fuse_tc(a0_ref, a1_ref, x_ref, d0_ref, d1_ref, w_ref, b_ref, o_ref):
    h = a0_ref[0] + a1_ref[0] + x_ref[...]
    deg = d0_ref[...] + d1_ref[...] + 1.0
    p = jnp.dot(h, w_ref[...], preferred_element_type=jnp.float32)
    o_ref[...] = p / deg + b_ref[...]


def _make_sc_scatter(n_pad, chunks_per_w, f):
    rows_per_tile = n_pad // 16
    zero_blks = rows_per_tile // CHUNK
    cpp = -(-chunks_per_w // 2)
    mesh = plsc.VectorSubcoreMesh(core_axis_name="c", subcore_axis_name="s")

    @functools.partial(
        pl.kernel,
        mesh=mesh,
        out_type=[
            jax.ShapeDtypeStruct((2, n_pad, f), jnp.float32),
            jax.ShapeDtypeStruct((2 * n_pad,), jnp.float32),
        ],
        scratch_types=[
            pltpu.VMEM((cpp, CHUNK), jnp.int32),
            pltpu.VMEM((cpp, CHUNK), jnp.int32),
            pltpu.VMEM((CHUNK, f), jnp.float32),
            pltpu.VMEM((CHUNK, f), jnp.float32),
            pltpu.VMEM((CHUNK,), jnp.float32),
            pltpu.VMEM((rows_per_tile,), jnp.float32),
            pltpu.VMEM_SHARED((n_pad, f), jnp.float32),
            pltpu.VMEM_SHARED((n_pad,), jnp.float32),
            pltpu.SemaphoreType.DMA,
            pltpu.SemaphoreType.DMA,
        ],
    )
    def sc_scatter(src_hbm, dst_hbm, x_hbm,
                   acc_out, deg_out,
                   src_v, dst_v, buf_a, buf_b, ones_v, zdeg_v,
                   acc_sh, deg_sh, sem_a, sem_b):
        c = lax.axis_index("c")
        s = lax.axis_index("s")
        wid = c * 16 + s
        row0 = s * rows_per_tile
        sl = pl.ds(row0, rows_per_tile)

        def zrow(i, carry):
            for k in range(f // 16):
                buf_a[i, pl.ds(k * 16, 16)] = jnp.zeros((16,), jnp.float32)
            return carry
        lax.fori_loop(0, CHUNK, zrow, 0)
        for i in range(rows_per_tile // 16):
            zdeg_v[pl.ds(i * 16, 16)] = jnp.zeros((16,), jnp.float32)
        for i in range(CHUNK // 16):
            ones_v[pl.ds(i * 16, 16)] = jnp.ones((16,), jnp.float32)
        for t in range(zero_blks):
            pltpu.sync_copy(buf_a, acc_sh.at[pl.ds(row0 + t * CHUNK, CHUNK)])
        pltpu.sync_copy(zdeg_v, deg_sh.at[sl])

        plsc.subcore_barrier()

        for p in range(2):
            lo = p * cpp
            cnt = min(cpp, chunks_per_w - lo)
            psl = pl.ds(0, cnt)
            pltpu.sync_copy(src_hbm.at[wid, pl.ds(lo, cnt)], src_v.at[psl])
            pltpu.sync_copy(dst_hbm.at[wid, pl.ds(lo, cnt)], dst_v.at[psl])

            pltpu.async_copy(x_hbm.at[src_v.at[0]], buf_a, sem_a)

            def body(j, carry):
                def step(buf_p, sem_p, buf_q, sem_q):
                    @pl.when(j + 1 < cnt)
                    def _():
                        pltpu.async_copy(
                            x_hbm.at[src_v.at[j + 1]], buf_q, sem_q)
                    pltpu.make_async_copy(
                        x_hbm.at[src_v.at[j]], buf_p, sem_p).wait()
                    pltpu.sync_copy(buf_p, acc_sh.at[dst_v.at[j]], add=True)
                    pltpu.sync_copy(ones_v, deg_sh.at[dst_v.at[j]], add=True)

                @pl.when(j % 2 == 0)
                def _():
                    step(buf_a, sem_a, buf_b, sem_b)

                @pl.when(j % 2 == 1)
                def _():
                    step(buf_b, sem_b, buf_a, sem_a)
                return carry

            lax.fori_loop(0, cnt, body, 0)

        plsc.subcore_barrier()

        pltpu.sync_copy(acc_sh.at[sl], acc_out.at[c, sl])
        pltpu.sync_copy(deg_sh.at[sl],
                        deg_out.at[pl.ds(c * n_pad + row0, rows_per_tile)])

    return sc_scatter


def kernel(features, edge_index, W_neigh, b):
    n, f = features.shape
    d = W_neigh.shape[1]
    e = edge_index.shape[1]

    n_pad = (n // 2048 + 1) * 2048
    chunks_per_w = -(-e // (NW * CHUNK))
    e_pad = NW * chunks_per_w * CHUNK

    src = edge_index[0].astype(jnp.int32)
    dst = edge_index[1].astype(jnp.int32)
    pad = e_pad - e
    spread = jnp.arange(pad, dtype=jnp.int32)
    src_p = jnp.concatenate([src, spread % n])
    dst_p = jnp.concatenate([dst, n + spread % (n_pad - n)])
    src_r = src_p.reshape(NW, chunks_per_w, CHUNK)
    dst_r = dst_p.reshape(NW, chunks_per_w, CHUNK)

    acc_part, deg_part = _make_sc_scatter(n_pad, chunks_per_w, f)(
        src_r, dst_r, features)

    blk = 1024
    grid = -(-n // blk)
    deg2 = deg_part.reshape(2 * n_pad, 1)
    b2 = b.reshape(1, d)
    out = pl.pallas_call(
        _fuse_tc,
        grid=(grid,),
        in_specs=[
            pl.BlockSpec((1, blk, f), lambda i: (0, i, 0)),
            pl.BlockSpec((1, blk, f), lambda i: (1, i, 0)),
            pl.BlockSpec((blk, f), lambda i: (i, 0)),
            pl.BlockSpec((blk, 1), lambda i: (i, 0)),
            pl.BlockSpec((blk, 1), lambda i: (n_pad // blk + i, 0)),
            pl.BlockSpec((f, d), lambda i: (0, 0)),
            pl.BlockSpec((1, d), lambda i: (0, 0)),
        ],
        out_specs=pl.BlockSpec((blk, d), lambda i: (i, 0)),
        out_shape=jax.ShapeDtypeStruct((n, d), jnp.float32),
    )(acc_part, acc_part, features, deg2, deg2, W_neigh, b2)
    return out
